# Initial kernel scaffold; baseline (speedup 1.0000x reference)
import jax
import jax.numpy as jnp
from jax.experimental import pallas as pl


def kernel(nodes, to_neighs, table):
    raise NotImplementedError("write your pallas kernel here")


# speedup vs baseline: 2.8614x; 2.8614x over previous
import functools

import jax
import jax.numpy as jnp
from jax import lax
from jax.experimental import pallas as pl
from jax.experimental.pallas import tpu as pltpu
from jax.experimental.pallas import tpu_sc as plsc

_NC = 2
_NS = 16
_NW = _NC * _NS
_L = 16


@functools.partial(jax.jit, static_argnames=("n_chunks", "c_nodes", "s"))
def _sc_mean_gather(idx, table, *, n_chunks, c_nodes, s):
    rows = c_nodes * s
    _, d = table.shape
    b_pad = _NW * n_chunks * c_nodes
    scale = jnp.float32(1.0 / (float(s) + 1e-15))

    mesh = plsc.VectorSubcoreMesh(core_axis_name="c", subcore_axis_name="s",
                                  num_cores=_NC, num_subcores=_NS)

    @functools.partial(
        pl.kernel,
        out_type=jax.ShapeDtypeStruct((b_pad, d), jnp.float32),
        mesh=mesh,
        scratch_types=[
            pltpu.VMEM((rows,), jnp.int32),
            pltpu.VMEM((rows, d), jnp.float32),
            pltpu.VMEM((c_nodes, d), jnp.float32),
        ],
    )
    def k(idx_hbm, table_hbm, out_hbm, idx_v, rows_v, out_v):
        wid = lax.axis_index("c") * _NS + lax.axis_index("s")

        @pl.loop(0, n_chunks)
        def _chunk(chunk):
            cbase = wid * n_chunks + chunk
            pltpu.sync_copy(idx_hbm.at[pl.ds(cbase * rows, rows)], idx_v)
            pltpu.sync_copy(table_hbm.at[idx_v], rows_v)

            @pl.loop(0, c_nodes)
            def _node(n):
                base = n * s
                for c in range(d // _L):
                    sl = pl.ds(c * _L, _L)
                    acc = rows_v[base, sl]
                    for kk in range(1, s):
                        acc = acc + rows_v[base + kk, sl]
                    out_v[n, sl] = acc * scale

            pltpu.sync_copy(out_v, out_hbm.at[pl.ds(cbase * c_nodes, c_nodes)])

    return k(idx, table)


def kernel(nodes, to_neighs, table):
    b, s = to_neighs.shape
    c_nodes = 8
    per_step = _NW * c_nodes
    n_chunks = -(-b // per_step)
    b_pad = n_chunks * per_step
    idx = to_neighs.reshape(-1)
    if b_pad != b:
        idx = jnp.pad(idx, (0, (b_pad - b) * s))
    out = _sc_mean_gather(idx, table, n_chunks=n_chunks, c_nodes=c_nodes, s=s)
    return out[:b]


# speedup vs baseline: 5.4202x; 1.8942x over previous
import functools

import jax
import jax.numpy as jnp
from jax import lax
from jax.experimental import pallas as pl
from jax.experimental.pallas import tpu as pltpu
from jax.experimental.pallas import tpu_sc as plsc

_NC = 2
_NS = 16
_NW = _NC * _NS
_L = 16


@functools.partial(jax.jit, static_argnames=("total_chunks", "c_nodes", "s"))
def _sc_mean_gather(idx, table, *, total_chunks, c_nodes, s):
    rows = c_nodes * s
    _, d = table.shape
    b_out = total_chunks * c_nodes
    t_max = -(-total_chunks // _NW)
    if t_max % 2:
        t_max += 1
    scale = jnp.float32(1.0 / (float(s) + 1e-15))

    mesh = plsc.VectorSubcoreMesh(core_axis_name="c", subcore_axis_name="s",
                                  num_cores=_NC, num_subcores=_NS)

    @functools.partial(
        pl.kernel,
        out_type=jax.ShapeDtypeStruct((b_out, d), jnp.float32),
        mesh=mesh,
        scratch_types=[
            pltpu.VMEM((t_max * rows,), jnp.int32),
            pltpu.VMEM((rows, d), jnp.float32),
            pltpu.VMEM((rows, d), jnp.float32),
            pltpu.VMEM((c_nodes, d), jnp.float32),
            pltpu.VMEM((c_nodes, d), jnp.float32),
            pltpu.SemaphoreType.DMA,
            pltpu.SemaphoreType.DMA,
            pltpu.SemaphoreType.DMA,
            pltpu.SemaphoreType.DMA,
        ],
    )
    def k(idx_hbm, table_hbm, out_hbm, idx_v, rows_v0, rows_v1,
          out_v0, out_v1, gsem0, gsem1, osem0, osem1):
        rows_v = (rows_v0, rows_v1)
        out_v = (out_v0, out_v1)
        gsem = (gsem0, gsem1)
        osem = (osem0, osem1)

        wid = lax.axis_index("c") * _NS + lax.axis_index("s")
        start_w = (wid * total_chunks) // _NW
        n_w = ((wid + 1) * total_chunks) // _NW - start_w
        nm1 = n_w - 1

        pltpu.sync_copy(idx_hbm.at[pl.ds(start_w * rows, t_max * rows)],
                        idx_v)

        def gather(step_lc, b):
            return pltpu.make_async_copy(
                table_hbm.at[idx_v.at[pl.ds(step_lc * rows, rows)]],
                rows_v[b], gsem[b])

        def out_store(step_lc, b):
            return pltpu.make_async_copy(
                out_v[b],
                out_hbm.at[pl.ds((start_w + step_lc) * c_nodes, c_nodes)],
                osem[b])

        for b in (0, 1):
            gather(lax.min(jnp.int32(b), nm1), b).start()

        @pl.loop(0, t_max // 2)
        def _steps(t):
            for b in (0, 1):
                i = 2 * t + b
                lc = lax.min(i, nm1)
                gather(lc, b).wait()

                @pl.when(t >= 1)
                def _():
                    out_store(lax.min(i - 2, nm1), b).wait()

                rv, ov = rows_v[b], out_v[b]
                for n in range(c_nodes):
                    for c in range(d // _L):
                        sl = pl.ds(c * _L, _L)
                        acc = rv[n * s, sl]
                        for kk in range(1, s):
                            acc = acc + rv[n * s + kk, sl]
                        ov[n, sl] = acc * scale

                out_store(lc, b).start()
                gather(lax.min(i + 2, nm1), b).start()

        for b in (0, 1):
            gather(nm1, b).wait()
            out_store(nm1, b).wait()

    return k(idx, table)


def kernel(nodes, to_neighs, table):
    b, s = to_neighs.shape
    c_nodes = 8
    total_chunks = -(-b // c_nodes)
    idx = to_neighs.reshape(-1)
    if total_chunks * c_nodes != b:
        idx = jnp.pad(idx, (0, (total_chunks * c_nodes - b) * s))
    t_max = -(-total_chunks // _NW)
    if t_max % 2:
        t_max += 1
    needed = (((_NW - 1) * total_chunks) // _NW + t_max) * c_nodes * s
    if needed > idx.shape[0]:
        idx = jnp.pad(idx, (0, needed - idx.shape[0]))
    out = _sc_mean_gather(idx, table, total_chunks=total_chunks,
                          c_nodes=c_nodes, s=s)
    return out[:b] if total_chunks * c_nodes != b else out
